# baseline (speedup 1.0000x reference)
import functools

import jax
import jax.numpy as jnp
from jax import lax
from jax.experimental import pallas as pl
from jax.experimental.pallas import tpu as pltpu
from jax.experimental.pallas import tpu_sc as plsc

VOCAB = 1000000
EMBED_DIM = 64
BATCH = 16384
SEQ = 50

_B = BATCH * SEQ
_D = EMBED_DIM

_info = plsc.get_sparse_core_info()
_NC = _info.num_cores
_NS = _info.num_subcores
_NW = _NC * _NS
_C = 256
_BPW = BATCH // _NW
_HPW = _BPW // _C
_NCHUNK = SEQ * _HPW


def _make_gather():
    mesh = plsc.VectorSubcoreMesh(core_axis_name="c", subcore_axis_name="s")

    @functools.partial(
        pl.kernel,
        mesh=mesh,
        out_type=jax.ShapeDtypeStruct((SEQ * 8 * 128 * 1024,), jnp.float32),
        compiler_params=pltpu.CompilerParams(
            use_tc_tiling_on_sc=False, needs_layout_passes=False),
        scratch_types=[
            pltpu.VMEM((_C,), jnp.int32),
            pltpu.VMEM((_C,), jnp.int32),
            pltpu.VMEM((_C, _D), jnp.float32),
            pltpu.VMEM((_C, _D), jnp.float32),
            pltpu.VMEM((16384,), jnp.float32),
            pltpu.VMEM((16384,), jnp.float32),
            pltpu.SemaphoreType.DMA,
            pltpu.SemaphoreType.DMA,
            pltpu.SemaphoreType.DMA,
            pltpu.SemaphoreType.DMA,
        ],
    )
    def gather_kernel(table_hbm, idx_hbm, out_hbm,
                      idx0, idx1, rows0, rows1, t0, t1,
                      g0, g1, o0, o1):
        wid = lax.axis_index("s") * _NC + lax.axis_index("c")
        bufs = ((idx0, rows0, t0, g0, o0), (idx1, rows1, t1, g1, o1))

        iota = lax.iota(jnp.int32, 16)
        dpat = [((k * 16 + iota) >> 3) * 2048 + ((k * 16 + iota) & 7) * 128
                for k in range(4)]

        def idx_off(m):
            s = m // _HPW
            h = m % _HPW
            return s * BATCH + wid * _BPW + h * _C

        def out_base(m):
            s = m // _HPW
            h = m % _HPW
            j0 = wid * (_BPW // 128) + h * (_C // 128)
            return (s * 8) * 131072 + j0 * 1024

        def out_copies(m, t_v, osem):
            base = out_base(m)
            return [
                pltpu.make_async_copy(
                    t_v.at[pl.ds(i * 2048, 2048)],
                    out_hbm.at[pl.ds(base + i * 131072, 2048)],
                    osem)
                for i in range(8)
            ]

        def start_chunk(m, idx_v, rows_v, gsem):
            pltpu.sync_copy(idx_hbm.at[pl.ds(idx_off(m), _C)], idx_v)
            pltpu.make_async_copy(table_hbm.at[idx_v], rows_v, gsem).start()

        for p in (0, 1):
            idx_v, rows_v, t_v, gsem, osem = bufs[p]
            start_chunk(p, idx_v, rows_v, gsem)

        def loop_body(n, carry):
            for p in (0, 1):
                idx_v, rows_v, t_v, gsem, osem = bufs[p]
                m = 2 * n + p
                pltpu.make_async_copy(table_hbm.at[idx_v], rows_v, gsem).wait()

                def tr_body(bb, c, rows_v=rows_v, t_v=t_v):
                    for u in range(4):
                        b = bb * 4 + u
                        for jj in (0, 1):
                            o = jj * 1024 + b
                            for k in range(4):
                                vals = rows_v[jj * 128 + b, pl.ds(k * 16, 16)]
                                plsc.store_scatter(t_v, [dpat[k] + o], vals)
                    return c

                lax.fori_loop(0, 32, tr_body, 0)

                @pl.when(n >= 1)
                def _():
                    for c in out_copies(m - 2, t_v, osem):
                        c.wait()

                for c in out_copies(m, t_v, osem):
                    c.start()

                @pl.when(n < _NCHUNK // 2 - 1)
                def _():
                    start_chunk(m + 2, idx_v, rows_v, gsem)
            return carry

        lax.fori_loop(0, _NCHUNK // 2, loop_body, 0)

        for p in (0, 1):
            idx_v, rows_v, t_v, gsem, osem = bufs[p]
            m = _NCHUNK - 2 + p
            for c in out_copies(m, t_v, osem):
                c.wait()

    return gather_kernel


_gather = _make_gather()


def kernel(indices, table):
    idx_flat = indices.T.reshape(_B).astype(jnp.int32)
    out = _gather(table, idx_flat)
    out5 = out.reshape(SEQ, 8, 128, 8, 128)
    return out5.transpose(2, 4, 0, 1, 3).reshape(BATCH, SEQ, _D)


# speedup vs baseline: 1.0281x; 1.0281x over previous
import functools

import jax
import jax.numpy as jnp
from jax import lax
from jax.experimental import pallas as pl
from jax.experimental.pallas import tpu as pltpu
from jax.experimental.pallas import tpu_sc as plsc

VOCAB = 1000000
EMBED_DIM = 64
BATCH = 16384
SEQ = 50

_B = BATCH * SEQ
_D = EMBED_DIM

_info = plsc.get_sparse_core_info()
_NC = _info.num_cores
_NS = _info.num_subcores
_NW = _NC * _NS
_C = 256
_BPW = BATCH // _NW
_HPW = _BPW // _C
_NCHUNK = SEQ * _HPW


def _make_gather():
    mesh = plsc.VectorSubcoreMesh(core_axis_name="c", subcore_axis_name="s")

    @functools.partial(
        pl.kernel,
        mesh=mesh,
        out_type=jax.ShapeDtypeStruct((SEQ * 8 * 128 * 1024,), jnp.float32),
        compiler_params=pltpu.CompilerParams(
            use_tc_tiling_on_sc=False, needs_layout_passes=False),
        scratch_types=[
            pltpu.VMEM((SEQ, _BPW), jnp.int32),
            pltpu.VMEM((_C, _D), jnp.float32),
            pltpu.VMEM((_C, _D), jnp.float32),
            pltpu.VMEM((16384,), jnp.float32),
            pltpu.VMEM((16384,), jnp.float32),
            pltpu.SemaphoreType.DMA,
            pltpu.SemaphoreType.DMA,
            pltpu.SemaphoreType.DMA,
            pltpu.SemaphoreType.DMA,
            pltpu.SemaphoreType.DMA,
            pltpu.SemaphoreType.DMA,
        ],
    )
    def gather_kernel(table_hbm, idx_hbm, out_hbm,
                      idx_all, rows0, rows1, t0, t1,
                      g0a, g0b, g1a, g1b, o0, o1):
        wid = lax.axis_index("s") * _NC + lax.axis_index("c")
        pltpu.sync_copy(idx_hbm.at[:, pl.ds(wid * _BPW, _BPW)], idx_all)
        bufs = ((rows0, t0, g0a, g0b, o0), (rows1, t1, g1a, g1b, o1))

        iota = lax.iota(jnp.int32, 16)
        dpat = [((k * 16 + iota) >> 3) * 2048 + ((k * 16 + iota) & 7) * 128
                for k in range(4)]

        def out_base(m):
            s = m // _HPW
            h = m % _HPW
            j0 = wid * (_BPW // 128) + h * (_C // 128)
            return (s * 8) * 131072 + j0 * 1024

        def out_copies(m, t_v, osem):
            base = out_base(m)
            return [
                pltpu.make_async_copy(
                    t_v.at[pl.ds(i * 2048, 2048)],
                    out_hbm.at[pl.ds(base + i * 131072, 2048)],
                    osem)
                for i in range(8)
            ]

        def start_chunk(m, rows_v, ga, gb):
            s = m // _HPW
            col = (m % _HPW) * _C
            pltpu.make_async_copy(
                table_hbm.at[idx_all.at[s, pl.ds(col, 128)]],
                rows_v.at[pl.ds(0, 128)], ga).start()
            pltpu.make_async_copy(
                table_hbm.at[idx_all.at[s, pl.ds(col + 128, 128)]],
                rows_v.at[pl.ds(128, 128)], gb).start()

        def wait_chunk(m, rows_v, ga, gb):
            s = m // _HPW
            col = (m % _HPW) * _C
            pltpu.make_async_copy(
                table_hbm.at[idx_all.at[s, pl.ds(col, 128)]],
                rows_v.at[pl.ds(0, 128)], ga).wait()
            pltpu.make_async_copy(
                table_hbm.at[idx_all.at[s, pl.ds(col + 128, 128)]],
                rows_v.at[pl.ds(128, 128)], gb).wait()

        for p in (0, 1):
            rows_v, t_v, ga, gb, osem = bufs[p]
            start_chunk(p, rows_v, ga, gb)

        def loop_body(n, carry):
            for p in (0, 1):
                rows_v, t_v, ga, gb, osem = bufs[p]
                m = 2 * n + p
                wait_chunk(m, rows_v, ga, gb)

                def tr_body(bb, c, rows_v=rows_v, t_v=t_v):
                    for u in range(4):
                        b = bb * 4 + u
                        for jj in (0, 1):
                            o = jj * 1024 + b
                            for k in range(4):
                                vals = rows_v[jj * 128 + b, pl.ds(k * 16, 16)]
                                plsc.store_scatter(t_v, [dpat[k] + o], vals)
                    return c

                lax.fori_loop(0, 32, tr_body, 0)

                @pl.when(n >= 1)
                def _():
                    for c in out_copies(m - 2, t_v, osem):
                        c.wait()

                for c in out_copies(m, t_v, osem):
                    c.start()

                @pl.when(n < _NCHUNK // 2 - 1)
                def _():
                    start_chunk(m + 2, rows_v, ga, gb)
            return carry

        lax.fori_loop(0, _NCHUNK // 2, loop_body, 0)

        for p in (0, 1):
            rows_v, t_v, ga, gb, osem = bufs[p]
            m = _NCHUNK - 2 + p
            for c in out_copies(m, t_v, osem):
                c.wait()

    return gather_kernel


_gather = _make_gather()


def kernel(indices, table):
    idx_sm = indices.T.astype(jnp.int32)
    out = _gather(table, idx_sm)
    out5 = out.reshape(SEQ, 8, 128, 8, 128)
    return out5.transpose(2, 4, 0, 1, 3).reshape(BATCH, SEQ, _D)


# speedup vs baseline: 1.0291x; 1.0009x over previous
import functools

import jax
import jax.numpy as jnp
from jax import lax
from jax.experimental import pallas as pl
from jax.experimental.pallas import tpu as pltpu
from jax.experimental.pallas import tpu_sc as plsc

VOCAB = 1000000
EMBED_DIM = 64
BATCH = 16384
SEQ = 50

_B = BATCH * SEQ
_D = EMBED_DIM

_info = plsc.get_sparse_core_info()
_NC = _info.num_cores
_NS = _info.num_subcores
_NW = _NC * _NS
_C = 256
_BPW = BATCH // _NW


def _make_gather():
    mesh = plsc.VectorSubcoreMesh(core_axis_name="c", subcore_axis_name="s")

    @functools.partial(
        pl.kernel,
        mesh=mesh,
        out_type=jax.ShapeDtypeStruct((SEQ * 8 * 128 * 1024,), jnp.float32),
        compiler_params=pltpu.CompilerParams(
            use_tc_tiling_on_sc=False, needs_layout_passes=False),
        scratch_types=[
            pltpu.VMEM((SEQ, _BPW), jnp.int32),
            pltpu.VMEM((_C, _D), jnp.float32),
            pltpu.VMEM((_C, _D), jnp.float32),
            pltpu.VMEM((8 * 4096,), jnp.float32),
            pltpu.VMEM((8 * 4096,), jnp.float32),
            pltpu.SemaphoreType.DMA,
            pltpu.SemaphoreType.DMA,
            pltpu.SemaphoreType.DMA,
            pltpu.SemaphoreType.DMA,
        ],
    )
    def gather_kernel(table_hbm, idx_hbm, out_hbm,
                      idx_all, rowsA, rowsB, tP0, tP1,
                      gA, gB, oP0, oP1):
        wid = lax.axis_index("s") * _NC + lax.axis_index("c")
        pltpu.sync_copy(idx_hbm.at[:, pl.ds(wid * _BPW, _BPW)], idx_all)

        rows_bufs = ((rowsA, gA), (rowsB, gB))
        t_bufs = ((tP0, oP0), (tP1, oP1))

        iota = lax.iota(jnp.int32, 16)
        dpat = [((k * 16 + iota) >> 3) * 4096 + ((k * 16 + iota) & 7) * 128
                for k in range(4)]

        def gather_copy(s, h):
            rows_v, gsem = rows_bufs[h]
            return pltpu.make_async_copy(
                table_hbm.at[idx_all.at[s, pl.ds(h * _C, _C)]], rows_v, gsem)

        def out_copies(s, t_v, osem):
            base = (s * 8) * 131072 + (wid * 4) * 1024
            return [
                pltpu.make_async_copy(
                    t_v.at[pl.ds(i * 4096, 4096)],
                    out_hbm.at[pl.ds(base + i * 131072, 4096)],
                    osem)
                for i in range(8)
            ]

        def transpose_half(h, rows_v, t_v):
            def tr_body(bb, c):
                for u in range(4):
                    b = bb * 4 + u
                    for jj in (0, 1):
                        o = h * 2048 + jj * 1024 + b
                        for k in range(4):
                            vals = rows_v[jj * 128 + b, pl.ds(k * 16, 16)]
                            plsc.store_scatter(t_v, [dpat[k] + o], vals)
                return c

            lax.fori_loop(0, 32, tr_body, 0)

        for h in (0, 1):
            gather_copy(0, h).start()

        def loop_body(n, carry):
            for p in (0, 1):
                s = 2 * n + p
                t_v, osem = t_bufs[p]

                @pl.when(n >= 1)
                def _():
                    for c in out_copies(s - 2, t_v, osem):
                        c.wait()

                for h in (0, 1):
                    rows_v, gsem = rows_bufs[h]
                    gather_copy(s, h).wait()
                    transpose_half(h, rows_v, t_v)
                    @pl.when(s < SEQ - 1)
                    def _():
                        gather_copy(s + 1, h).start()

                for c in out_copies(s, t_v, osem):
                    c.start()
            return carry

        lax.fori_loop(0, SEQ // 2, loop_body, 0)

        for p in (0, 1):
            s = SEQ - 2 + p
            t_v, osem = t_bufs[p]
            for c in out_copies(s, t_v, osem):
                c.wait()

    return gather_kernel


_gather = _make_gather()


def kernel(indices, table):
    idx_sm = indices.T.astype(jnp.int32)
    out = _gather(table, idx_sm)
    out5 = out.reshape(SEQ, 8, 128, 8, 128)
    return out5.transpose(2, 4, 0, 1, 3).reshape(BATCH, SEQ, _D)


# speedup vs baseline: 1.2467x; 1.2115x over previous
import functools

import jax
import jax.numpy as jnp
from jax import lax
from jax.experimental import pallas as pl
from jax.experimental.pallas import tpu as pltpu
from jax.experimental.pallas import tpu_sc as plsc

VOCAB = 1000000
EMBED_DIM = 64
BATCH = 16384
SEQ = 50

_B = BATCH * SEQ
_D = EMBED_DIM

_info = plsc.get_sparse_core_info()
_NC = _info.num_cores
_NS = _info.num_subcores
_NW = _NC * _NS
_C = 256
_BPW = BATCH // _NW


def _make_gather():
    mesh = plsc.VectorSubcoreMesh(core_axis_name="c", subcore_axis_name="s")

    @functools.partial(
        pl.kernel,
        mesh=mesh,
        out_type=jax.ShapeDtypeStruct((SEQ * 8 * 128 * 1024,), jnp.float32),
        compiler_params=pltpu.CompilerParams(
            use_tc_tiling_on_sc=False, needs_layout_passes=False),
        scratch_types=[
            pltpu.VMEM((SEQ, _BPW), jnp.int32),
            pltpu.VMEM((_C, _D), jnp.float32),
            pltpu.VMEM((_C, _D), jnp.float32),
            pltpu.VMEM((8 * 4096,), jnp.float32),
            pltpu.VMEM((8 * 4096,), jnp.float32),
            pltpu.SemaphoreType.DMA,
            pltpu.SemaphoreType.DMA,
            pltpu.SemaphoreType.DMA,
            pltpu.SemaphoreType.DMA,
        ],
    )
    def gather_kernel(table_hbm, idx_hbm, out_hbm,
                      idx_all, rowsA, rowsB, tP0, tP1,
                      gA, gB, oP0, oP1):
        wid = lax.axis_index("s") * _NC + lax.axis_index("c")
        pltpu.sync_copy(idx_hbm.at[:, pl.ds(wid * _BPW, _BPW)], idx_all)

        rows_bufs = ((rowsA, gA), (rowsB, gB))
        t_bufs = ((tP0, oP0), (tP1, oP1))

        iota = lax.iota(jnp.int32, 16)
        dpat = [((k * 16 + iota) >> 3) * 4096 + ((k * 16 + iota) & 7) * 128
                for k in range(4)]

        def gather_copy(s, h):
            rows_v, gsem = rows_bufs[h]
            return pltpu.make_async_copy(
                table_hbm.at[idx_all.at[s, pl.ds(h * _C, _C)]], rows_v, gsem)

        def out_copies(s, t_v, osem):
            base = (s * 8) * 131072 + (wid * 4) * 1024
            return [
                pltpu.make_async_copy(
                    t_v.at[pl.ds(i * 4096, 4096)],
                    out_hbm.at[pl.ds(base + i * 131072, 4096)],
                    osem)
                for i in range(8)
            ]

        def transpose_half(h, rows_v, t_v):
            @plsc.parallel_loop(0, 32)
            def tr_body(bb):
                for u in range(4):
                    b = bb * 4 + u
                    for jj in (0, 1):
                        o = h * 2048 + jj * 1024 + b
                        for k in range(4):
                            vals = rows_v[jj * 128 + b, pl.ds(k * 16, 16)]
                            plsc.store_scatter(t_v, [dpat[k] + o], vals)

        for h in (0, 1):
            gather_copy(0, h).start()

        def loop_body(n, carry):
            for p in (0, 1):
                s = 2 * n + p
                t_v, osem = t_bufs[p]

                @pl.when(n >= 1)
                def _():
                    for c in out_copies(s - 2, t_v, osem):
                        c.wait()

                for h in (0, 1):
                    rows_v, gsem = rows_bufs[h]
                    gather_copy(s, h).wait()
                    transpose_half(h, rows_v, t_v)
                    @pl.when(s < SEQ - 1)
                    def _():
                        gather_copy(s + 1, h).start()

                for c in out_copies(s, t_v, osem):
                    c.start()
            return carry

        lax.fori_loop(0, SEQ // 2, loop_body, 0)

        for p in (0, 1):
            s = SEQ - 2 + p
            t_v, osem = t_bufs[p]
            for c in out_copies(s, t_v, osem):
                c.wait()

    return gather_kernel


_gather = _make_gather()


def kernel(indices, table):
    idx_sm = indices.T.astype(jnp.int32)
    out = _gather(table, idx_sm)
    out5 = out.reshape(SEQ, 8, 128, 8, 128)
    return out5.transpose(2, 4, 0, 1, 3).reshape(BATCH, SEQ, _D)
